# baseline (speedup 1.0000x reference)
import jax
import jax.numpy as jnp
from jax.experimental import pallas as pl
from jax.experimental.pallas import tpu as pltpu

_BM = 2048
_D = 256


def _body(idx_ref, x_hbm, pos_ref, out_ref, pos_full, sem):
    i = pl.program_id(0)
    cp = pltpu.make_async_copy(
        x_hbm.at[pl.ds(i * _BM, _BM), :, :],
        out_ref.at[:, :, pl.ds(0, _D)],
        sem,
    )
    cp.start()

    @pl.when(i == 0)
    def _():
        row = idx_ref[0] % 8
        pos_full[...] = jnp.broadcast_to(
            pos_ref[0, pl.ds(row, 1), :], (_BM, _D))

    out_ref[:, 0, _D:2 * _D] = pos_full[...]
    cp.wait()


def kernel(x, pos_table, index):
    B, _, D = x.shape
    grid = B // _BM
    idx = jnp.asarray(index, jnp.int32).reshape(1)
    return pl.pallas_call(
        _body,
        grid_spec=pltpu.PrefetchScalarGridSpec(
            num_scalar_prefetch=1,
            grid=(grid,),
            in_specs=[
                pl.BlockSpec(memory_space=pltpu.HBM),
                pl.BlockSpec((1, 8, D), lambda i, s: (0, s[0] // 8, 0)),
            ],
            out_specs=pl.BlockSpec((_BM, 1, 2 * D), lambda i, s: (i, 0, 0)),
            scratch_shapes=[
                pltpu.VMEM((_BM, _D), jnp.float32),
                pltpu.SemaphoreType.DMA,
            ],
        ),
        out_shape=jax.ShapeDtypeStruct((B, 1, 2 * D), jnp.float32),
        compiler_params=pltpu.CompilerParams(
            dimension_semantics=("parallel",),
        ),
    )(idx, x, pos_table)


# speedup vs baseline: 1.4242x; 1.4242x over previous
import jax
import jax.numpy as jnp
from jax.experimental import pallas as pl
from jax.experimental.pallas import tpu as pltpu

_BM = 8192
_D = 256


def _body(idx_ref, x_hbm, pos_ref, out_ref, pos_full, sem):
    i = pl.program_id(0)
    cp = pltpu.make_async_copy(
        x_hbm.at[pl.ds(i * _BM, _BM), :, :],
        out_ref.at[:, :, pl.ds(0, _D)],
        sem,
    )
    cp.start()

    @pl.when(i == 0)
    def _():
        row = idx_ref[0] % 8
        pos_full[...] = jnp.broadcast_to(
            pos_ref[0, pl.ds(row, 1), :], (_BM, _D))

    out_ref[:, 0, _D:2 * _D] = pos_full[...]
    cp.wait()


def kernel(x, pos_table, index):
    B, _, D = x.shape
    grid = B // _BM
    idx = jnp.asarray(index, jnp.int32).reshape(1)
    return pl.pallas_call(
        _body,
        grid_spec=pltpu.PrefetchScalarGridSpec(
            num_scalar_prefetch=1,
            grid=(grid,),
            in_specs=[
                pl.BlockSpec(memory_space=pltpu.HBM),
                pl.BlockSpec((1, 8, D), lambda i, s: (0, s[0] // 8, 0)),
            ],
            out_specs=pl.BlockSpec((_BM, 1, 2 * D), lambda i, s: (i, 0, 0)),
            scratch_shapes=[
                pltpu.VMEM((_BM, _D), jnp.float32),
                pltpu.SemaphoreType.DMA,
            ],
        ),
        out_shape=jax.ShapeDtypeStruct((B, 1, 2 * D), jnp.float32),
        compiler_params=pltpu.CompilerParams(
            dimension_semantics=("parallel",),
        ),
    )(idx, x, pos_table)
